# Initial kernel scaffold; baseline (speedup 1.0000x reference)
import jax
import jax.numpy as jnp
from jax.experimental import pallas as pl


def kernel(features, edge_index, W1_self, W1_neigh, b1, W2_self, W2_neigh, b2, Wm1, bm1, Wm2, bm2):
    raise NotImplementedError("write your pallas kernel here")


# speedup vs baseline: 3.2163x; 3.2163x over previous
import functools

import jax
import jax.numpy as jnp
from jax import lax
from jax.experimental import pallas as pl
from jax.experimental.pallas import tpu as pltpu
from jax.experimental.pallas import tpu_sc as plsc

N_NODES = 10000
N_EDGES = 320000
D_FEAT = 128

NC = 2
NS = 16
NW = NC * NS

NPAD = 10240
EPAD = 327680
IDX_ROWS = EPAD // 128
ROWS_PER_W = IDX_ROWS // NW
KI = 2
IB = 40
ROWS_PER_SUB = NPAD // NS


def _make_sc_agg(with_deg):
    mesh = plsc.VectorSubcoreMesh(core_axis_name="c", subcore_axis_name="s")

    out_type = [jax.ShapeDtypeStruct((NC, NPAD, D_FEAT), jnp.float32)]
    scratch = [
        pltpu.VMEM((IB, 128), jnp.int32),
        pltpu.VMEM((IB, 128), jnp.int32),
        pltpu.VMEM((KI, 128, D_FEAT), jnp.float32),
        pltpu.VMEM_SHARED((NPAD, D_FEAT), jnp.float32),
        pltpu.SemaphoreType.DMA,
    ]
    if with_deg:
        out_type.append(jax.ShapeDtypeStruct((NC, NPAD), jnp.float32))
        scratch += [
            pltpu.VMEM((128,), jnp.float32),
            pltpu.VMEM_SHARED((NPAD,), jnp.float32),
        ]

    def body(table, src_r, dst_r, zeros, *rest):
        if with_deg:
            (zerosd, out, out_deg, sidx, didx, rows, acc, sem, ones,
             acc_deg) = rest
        else:
            out, sidx, didx, rows, acc, sem = rest

        c = lax.axis_index("c")
        s = lax.axis_index("s")
        wid = c * NS + s

        zsl = pl.ds(s * ROWS_PER_SUB, ROWS_PER_SUB)
        pltpu.sync_copy(zeros.at[zsl], acc.at[zsl])
        if with_deg:
            pltpu.sync_copy(zerosd.at[zsl], acc_deg.at[zsl])
            for i in range(8):
                ones[pl.ds(16 * i, 16)] = jnp.ones((16,), jnp.float32)
        plsc.subcore_barrier()

        base = wid * ROWS_PER_W

        def outer(ob, carry):
            r0 = base + ob * IB
            pltpu.sync_copy(src_r.at[pl.ds(r0, IB)], sidx)
            pltpu.sync_copy(dst_r.at[pl.ds(r0, IB)], didx)

            def step(g, c2):
                handles = [
                    pltpu.async_copy(
                        table.at[sidx.at[g * KI + j]], rows.at[j], sem)
                    for j in range(KI)
                ]
                for h in handles:
                    h.wait()
                for j in range(KI):
                    pltpu.sync_copy(
                        rows.at[j], acc.at[didx.at[g * KI + j]], add=True)
                    if with_deg:
                        pltpu.sync_copy(
                            ones, acc_deg.at[didx.at[g * KI + j]], add=True)
                return c2

            lax.fori_loop(0, IB // KI, step, carry)
            return carry

        lax.fori_loop(0, ROWS_PER_W // IB, outer, 0)

        plsc.subcore_barrier()
        pltpu.sync_copy(acc.at[zsl], out.at[c, zsl])
        if with_deg:
            pltpu.sync_copy(acc_deg.at[zsl], out_deg.at[c, zsl])

    return functools.partial(
        pl.kernel, mesh=mesh, out_type=out_type, scratch_types=scratch)(body)


_sc_agg_l1 = _make_sc_agg(with_deg=True)
_sc_agg_l2 = _make_sc_agg(with_deg=False)


ROW_BLK = 400
GRID = N_NODES // ROW_BLK


def _tc1_body(x_ref, sc_ref, deg_ref, ws_ref, wn_ref, b_ref, h_ref, inv_ref):
    agg = sc_ref[0] + sc_ref[1]
    deg = deg_ref[:, 0:1] + deg_ref[:, 1:2]
    inv = 1.0 / jnp.maximum(deg, 1.0)
    mean = agg * inv
    h = (jnp.dot(x_ref[...], ws_ref[...], preferred_element_type=jnp.float32)
         + jnp.dot(mean, wn_ref[...], preferred_element_type=jnp.float32)
         + b_ref[...])
    h_ref[...] = jax.nn.sigmoid(h)
    inv_ref[...] = jnp.broadcast_to(inv, (ROW_BLK, 8))


def _tc2_body(h1_ref, sc_ref, inv_ref, w2s_ref, w2n_ref, b2_ref,
              wm1_ref, bm1_ref, wm2_ref, bm2_ref, out_ref):
    agg = sc_ref[0] + sc_ref[1]
    inv = inv_ref[:, 0:1]
    h1 = h1_ref[...]
    h2 = jax.nn.sigmoid(
        jnp.dot(h1, w2s_ref[...], preferred_element_type=jnp.float32)
        + jnp.dot(agg * inv, w2n_ref[...], preferred_element_type=jnp.float32)
        + b2_ref[...])
    t = jnp.maximum(
        jnp.dot(h2, wm1_ref[...], preferred_element_type=jnp.float32)
        + bm1_ref[...], 0.0)
    out_ref[...] = (jnp.dot(t, wm2_ref[...], preferred_element_type=jnp.float32)
                    + bm2_ref[...])


def _full(shape):
    return pl.BlockSpec(shape, lambda i: tuple(0 for _ in shape))


def kernel(features, edge_index, W1_self, W1_neigh, b1, W2_self, W2_neigh, b2,
           Wm1, bm1, Wm2, bm2):
    src = jnp.asarray(edge_index[0], jnp.int32)
    dst = jnp.asarray(edge_index[1], jnp.int32)
    pad = EPAD - N_EDGES
    src_r = jnp.concatenate([src, jnp.zeros((pad,), jnp.int32)]).reshape(
        IDX_ROWS, 128)
    dst_r = jnp.concatenate([dst, jnp.full((pad,), N_NODES, jnp.int32)]).reshape(
        IDX_ROWS, 128)

    z2 = jnp.zeros((NPAD, D_FEAT), jnp.float32)
    zd = jnp.zeros((NPAD,), jnp.float32)

    sc1, deg2 = _sc_agg_l1(features, src_r, dst_r, z2, zd)
    deg_pair = deg2[:, :N_NODES].T

    h1, inv8 = pl.pallas_call(
        _tc1_body,
        grid=(GRID,),
        in_specs=[
            pl.BlockSpec((ROW_BLK, D_FEAT), lambda i: (i, 0)),
            pl.BlockSpec((NC, ROW_BLK, D_FEAT), lambda i: (0, i, 0)),
            pl.BlockSpec((ROW_BLK, NC), lambda i: (i, 0)),
            _full((D_FEAT, D_FEAT)),
            _full((D_FEAT, D_FEAT)),
            _full((1, D_FEAT)),
        ],
        out_specs=[
            pl.BlockSpec((ROW_BLK, D_FEAT), lambda i: (i, 0)),
            pl.BlockSpec((ROW_BLK, 8), lambda i: (i, 0)),
        ],
        out_shape=[
            jax.ShapeDtypeStruct((N_NODES, D_FEAT), jnp.float32),
            jax.ShapeDtypeStruct((N_NODES, 8), jnp.float32),
        ],
    )(features, sc1[:, :N_NODES], deg_pair, W1_self, W1_neigh,
      b1.reshape(1, D_FEAT))

    sc2, = _sc_agg_l2(h1, src_r, dst_r, z2)

    out = pl.pallas_call(
        _tc2_body,
        grid=(GRID,),
        in_specs=[
            pl.BlockSpec((ROW_BLK, D_FEAT), lambda i: (i, 0)),
            pl.BlockSpec((NC, ROW_BLK, D_FEAT), lambda i: (0, i, 0)),
            pl.BlockSpec((ROW_BLK, 8), lambda i: (i, 0)),
            _full((D_FEAT, D_FEAT)),
            _full((D_FEAT, D_FEAT)),
            _full((1, D_FEAT)),
            _full((D_FEAT, D_FEAT)),
            _full((1, D_FEAT)),
            _full((D_FEAT, 64)),
            _full((1, 64)),
        ],
        out_specs=pl.BlockSpec((ROW_BLK, 64), lambda i: (i, 0)),
        out_shape=jax.ShapeDtypeStruct((N_NODES, 64), jnp.float32),
    )(h1, sc2[:, :N_NODES], inv8, W2_self, W2_neigh, b2.reshape(1, D_FEAT),
      Wm1, bm1.reshape(1, D_FEAT), Wm2, bm2.reshape(1, 64))

    return out


# speedup vs baseline: 3.4327x; 1.0673x over previous
import functools

import jax
import jax.numpy as jnp
from jax import lax
from jax.experimental import pallas as pl
from jax.experimental.pallas import tpu as pltpu
from jax.experimental.pallas import tpu_sc as plsc

N_NODES = 10000
N_EDGES = 320000
D_FEAT = 128

NC = 2
NS = 16
NW = NC * NS

NPAD = 10240
EPAD = 327680
IDX_ROWS = EPAD // 128
ROWS_PER_W = IDX_ROWS // NW
KI = 2
IB = 40
ROWS_PER_SUB = NPAD // NS


def _make_sc_agg(with_deg):
    mesh = plsc.VectorSubcoreMesh(core_axis_name="c", subcore_axis_name="s")

    out_type = [jax.ShapeDtypeStruct((NC, NPAD, D_FEAT), jnp.float32)]
    scratch = [
        pltpu.VMEM((IB, 128), jnp.int32),
        pltpu.VMEM((IB, 128), jnp.int32),
        pltpu.VMEM((KI, 128, D_FEAT), jnp.float32),
        pltpu.VMEM_SHARED((NPAD, D_FEAT), jnp.float32),
        pltpu.SemaphoreType.DMA,
        pltpu.SemaphoreType.DMA,
        pltpu.SemaphoreType.DMA,
        pltpu.SemaphoreType.DMA,
    ]
    if with_deg:
        out_type.append(jax.ShapeDtypeStruct((NC, NPAD), jnp.float32))
        scratch += [
            pltpu.VMEM((128,), jnp.float32),
            pltpu.VMEM_SHARED((NPAD,), jnp.float32),
            pltpu.SemaphoreType.DMA,
        ]

    NP = IB // 2

    def body(table, src_r, dst_r, zeros, *rest):
        if with_deg:
            (zerosd, out, out_deg, sidx, didx, rows, acc, gs0, gs1, ss0, ss1,
             ones, acc_deg, dsem) = rest
        else:
            out, sidx, didx, rows, acc, gs0, gs1, ss0, ss1 = rest

        c = lax.axis_index("c")
        s = lax.axis_index("s")
        wid = c * NS + s

        gsem = (gs0, gs1)
        ssem = (ss0, ss1)

        zsl = pl.ds(s * ROWS_PER_SUB, ROWS_PER_SUB)
        pltpu.sync_copy(zeros.at[zsl], acc.at[zsl])
        if with_deg:
            pltpu.sync_copy(zerosd.at[zsl], acc_deg.at[zsl])
            for i in range(8):
                ones[pl.ds(16 * i, 16)] = jnp.ones((16,), jnp.float32)
        plsc.subcore_barrier()

        base = wid * ROWS_PER_W

        def gather(i, b):
            return pltpu.async_copy(table.at[sidx.at[i]], rows.at[b], gsem[b])

        def scatter(i, b):
            pltpu.async_copy(rows.at[b], acc.at[didx.at[i]], ssem[b], add=True)
            if with_deg:
                pltpu.async_copy(ones, acc_deg.at[didx.at[i]], dsem, add=True)

        def wait_gather(i, b):
            pltpu.make_async_copy(table.at[sidx.at[i]], rows.at[b],
                                  gsem[b]).wait()

        def wait_scatter(i, b):
            pltpu.make_async_copy(rows.at[b], acc.at[didx.at[i]],
                                  ssem[b]).wait()

        def outer(ob, carry):
            r0 = base + ob * IB
            pltpu.sync_copy(src_r.at[pl.ds(r0, IB)], sidx)
            pltpu.sync_copy(dst_r.at[pl.ds(r0, IB)], didx)

            gather(0, 0)
            gather(1, 1)

            def pair(gp, c2):
                i0 = gp * 2
                wait_gather(i0, 0)
                scatter(i0, 0)
                wait_gather(i0 + 1, 1)
                scatter(i0 + 1, 1)
                wait_scatter(i0, 0)
                gather(i0 + 2, 0)
                wait_scatter(i0 + 1, 1)
                gather(i0 + 3, 1)
                return c2

            lax.fori_loop(0, NP - 1, pair, carry)

            i0 = (NP - 1) * 2
            wait_gather(i0, 0)
            scatter(i0, 0)
            wait_gather(i0 + 1, 1)
            scatter(i0 + 1, 1)
            wait_scatter(i0, 0)
            wait_scatter(i0 + 1, 1)
            if with_deg:
                def drain(i, c3):
                    pltpu.make_async_copy(ones, acc_deg.at[didx.at[0]],
                                          dsem).wait()
                    return c3
                lax.fori_loop(0, IB, drain, carry)
            return carry

        lax.fori_loop(0, ROWS_PER_W // IB, outer, 0)

        plsc.subcore_barrier()
        pltpu.sync_copy(acc.at[zsl], out.at[c, zsl])
        if with_deg:
            pltpu.sync_copy(acc_deg.at[zsl], out_deg.at[c, zsl])

    return functools.partial(
        pl.kernel, mesh=mesh, out_type=out_type, scratch_types=scratch)(body)


_sc_agg_l1 = _make_sc_agg(with_deg=True)
_sc_agg_l2 = _make_sc_agg(with_deg=False)


ROW_BLK = 400
GRID = N_NODES // ROW_BLK


def _tc1_body(x_ref, sc_ref, deg_ref, ws_ref, wn_ref, b_ref, h_ref, inv_ref):
    agg = sc_ref[0] + sc_ref[1]
    deg = deg_ref[:, 0:1] + deg_ref[:, 1:2]
    inv = 1.0 / jnp.maximum(deg, 1.0)
    mean = agg * inv
    h = (jnp.dot(x_ref[...], ws_ref[...], preferred_element_type=jnp.float32)
         + jnp.dot(mean, wn_ref[...], preferred_element_type=jnp.float32)
         + b_ref[...])
    h_ref[...] = jax.nn.sigmoid(h)
    inv_ref[...] = jnp.broadcast_to(inv, (ROW_BLK, 8))


def _tc2_body(h1_ref, sc_ref, inv_ref, w2s_ref, w2n_ref, b2_ref,
              wm1_ref, bm1_ref, wm2_ref, bm2_ref, out_ref):
    agg = sc_ref[0] + sc_ref[1]
    inv = inv_ref[:, 0:1]
    h1 = h1_ref[...]
    h2 = jax.nn.sigmoid(
        jnp.dot(h1, w2s_ref[...], preferred_element_type=jnp.float32)
        + jnp.dot(agg * inv, w2n_ref[...], preferred_element_type=jnp.float32)
        + b2_ref[...])
    t = jnp.maximum(
        jnp.dot(h2, wm1_ref[...], preferred_element_type=jnp.float32)
        + bm1_ref[...], 0.0)
    out_ref[...] = (jnp.dot(t, wm2_ref[...], preferred_element_type=jnp.float32)
                    + bm2_ref[...])


def _full(shape):
    return pl.BlockSpec(shape, lambda i: tuple(0 for _ in shape))


def kernel(features, edge_index, W1_self, W1_neigh, b1, W2_self, W2_neigh, b2,
           Wm1, bm1, Wm2, bm2):
    src = jnp.asarray(edge_index[0], jnp.int32)
    dst = jnp.asarray(edge_index[1], jnp.int32)
    pad = EPAD - N_EDGES
    src_r = jnp.concatenate([src, jnp.zeros((pad,), jnp.int32)]).reshape(
        IDX_ROWS, 128)
    dst_r = jnp.concatenate([dst, jnp.full((pad,), N_NODES, jnp.int32)]).reshape(
        IDX_ROWS, 128)

    z2 = jnp.zeros((NPAD, D_FEAT), jnp.float32)
    zd = jnp.zeros((NPAD,), jnp.float32)

    sc1, deg2 = _sc_agg_l1(features, src_r, dst_r, z2, zd)
    deg_pair = deg2[:, :N_NODES].T

    h1, inv8 = pl.pallas_call(
        _tc1_body,
        grid=(GRID,),
        in_specs=[
            pl.BlockSpec((ROW_BLK, D_FEAT), lambda i: (i, 0)),
            pl.BlockSpec((NC, ROW_BLK, D_FEAT), lambda i: (0, i, 0)),
            pl.BlockSpec((ROW_BLK, NC), lambda i: (i, 0)),
            _full((D_FEAT, D_FEAT)),
            _full((D_FEAT, D_FEAT)),
            _full((1, D_FEAT)),
        ],
        out_specs=[
            pl.BlockSpec((ROW_BLK, D_FEAT), lambda i: (i, 0)),
            pl.BlockSpec((ROW_BLK, 8), lambda i: (i, 0)),
        ],
        out_shape=[
            jax.ShapeDtypeStruct((N_NODES, D_FEAT), jnp.float32),
            jax.ShapeDtypeStruct((N_NODES, 8), jnp.float32),
        ],
    )(features, sc1[:, :N_NODES], deg_pair, W1_self, W1_neigh,
      b1.reshape(1, D_FEAT))

    sc2, = _sc_agg_l2(h1, src_r, dst_r, z2)

    out = pl.pallas_call(
        _tc2_body,
        grid=(GRID,),
        in_specs=[
            pl.BlockSpec((ROW_BLK, D_FEAT), lambda i: (i, 0)),
            pl.BlockSpec((NC, ROW_BLK, D_FEAT), lambda i: (0, i, 0)),
            pl.BlockSpec((ROW_BLK, 8), lambda i: (i, 0)),
            _full((D_FEAT, D_FEAT)),
            _full((D_FEAT, D_FEAT)),
            _full((1, D_FEAT)),
            _full((D_FEAT, D_FEAT)),
            _full((1, D_FEAT)),
            _full((D_FEAT, 64)),
            _full((1, 64)),
        ],
        out_specs=pl.BlockSpec((ROW_BLK, 64), lambda i: (i, 0)),
        out_shape=jax.ShapeDtypeStruct((N_NODES, 64), jnp.float32),
    )(h1, sc2[:, :N_NODES], inv8, W2_self, W2_neigh, b2.reshape(1, D_FEAT),
      Wm1, bm1.reshape(1, D_FEAT), Wm2, bm2.reshape(1, 64))

    return out


# speedup vs baseline: 9.8458x; 2.8683x over previous
import functools

import jax
import jax.numpy as jnp
from jax import lax
from jax.experimental import pallas as pl
from jax.experimental.pallas import tpu as pltpu
from jax.experimental.pallas import tpu_sc as plsc

N_NODES = 10000
N_EDGES = 320000
D_FEAT = 128

NC = 2
NS = 16
NW = NC * NS

NPAD = 10240
EPAD = 327680
IDX_ROWS = EPAD // 128
ROWS_PER_W = IDX_ROWS // NW
KI = 2
IB = 40
ROWS_PER_SUB = NPAD // NS


def _make_sc_agg(with_deg):
    mesh = plsc.VectorSubcoreMesh(core_axis_name="c", subcore_axis_name="s")

    out_type = [jax.ShapeDtypeStruct((NC, NPAD, D_FEAT), jnp.float32)]
    scratch = [
        pltpu.VMEM((IB, 128), jnp.int32),
        pltpu.VMEM((IB, 128), jnp.int32),
        pltpu.VMEM((KI, 128, D_FEAT), jnp.float32),
        pltpu.VMEM_SHARED((NPAD, D_FEAT), jnp.float32),
        pltpu.SemaphoreType.DMA,
        pltpu.SemaphoreType.DMA,
        pltpu.SemaphoreType.DMA,
        pltpu.SemaphoreType.DMA,
    ]
    if with_deg:
        out_type.append(jax.ShapeDtypeStruct((NC, NPAD), jnp.float32))
        scratch += [
            pltpu.VMEM((128,), jnp.float32),
            pltpu.VMEM_SHARED((NPAD,), jnp.float32),
            pltpu.SemaphoreType.DMA,
        ]

    NP = IB // 2

    def body(table, src_r, dst_r, zeros, *rest):
        if with_deg:
            (zerosd, out, out_deg, sidx, didx, rows, acc, gs0, gs1, ss0, ss1,
             ones, acc_deg, dsem) = rest
        else:
            out, sidx, didx, rows, acc, gs0, gs1, ss0, ss1 = rest

        c = lax.axis_index("c")
        s = lax.axis_index("s")
        wid = c * NS + s

        gsem = (gs0, gs1)
        ssem = (ss0, ss1)

        zsl = pl.ds(s * ROWS_PER_SUB, ROWS_PER_SUB)
        pltpu.sync_copy(zeros.at[zsl], acc.at[zsl])
        if with_deg:
            pltpu.sync_copy(zerosd.at[zsl], acc_deg.at[zsl])
            for i in range(8):
                ones[pl.ds(16 * i, 16)] = jnp.ones((16,), jnp.float32)
        plsc.subcore_barrier()

        base = wid * ROWS_PER_W

        def gather(i, b):
            return pltpu.async_copy(table.at[sidx.at[i]], rows.at[b], gsem[b])

        def scatter(i, b):
            pltpu.async_copy(rows.at[b], acc.at[didx.at[i]], ssem[b], add=True)
            if with_deg:
                pltpu.async_copy(ones, acc_deg.at[didx.at[i]], dsem, add=True)

        def wait_gather(i, b):
            pltpu.make_async_copy(table.at[sidx.at[i]], rows.at[b],
                                  gsem[b]).wait()

        def wait_scatter(i, b):
            pltpu.make_async_copy(rows.at[b], acc.at[didx.at[i]],
                                  ssem[b]).wait()

        def outer(ob, carry):
            r0 = base + ob * IB
            pltpu.sync_copy(src_r.at[pl.ds(r0, IB)], sidx)
            pltpu.sync_copy(dst_r.at[pl.ds(r0, IB)], didx)

            gather(0, 0)
            gather(1, 1)

            def pair(gp, c2):
                i0 = gp * 2
                wait_gather(i0, 0)
                scatter(i0, 0)
                wait_gather(i0 + 1, 1)
                scatter(i0 + 1, 1)
                wait_scatter(i0, 0)
                gather(i0 + 2, 0)
                wait_scatter(i0 + 1, 1)
                gather(i0 + 3, 1)
                return c2

            lax.fori_loop(0, NP - 1, pair, carry)

            i0 = (NP - 1) * 2
            wait_gather(i0, 0)
            scatter(i0, 0)
            wait_gather(i0 + 1, 1)
            scatter(i0 + 1, 1)
            wait_scatter(i0, 0)
            wait_scatter(i0 + 1, 1)
            if with_deg:
                def drain(i, c3):
                    pltpu.make_async_copy(ones, acc_deg.at[didx.at[0]],
                                          dsem).wait()
                    return c3
                lax.fori_loop(0, IB, drain, carry)
            return carry

        lax.fori_loop(0, ROWS_PER_W // IB, outer, 0)

        plsc.subcore_barrier()
        pltpu.sync_copy(acc.at[zsl], out.at[c, zsl])
        if with_deg:
            pltpu.sync_copy(acc_deg.at[zsl], out_deg.at[c, zsl])

    return functools.partial(
        pl.kernel, mesh=mesh, out_type=out_type, scratch_types=scratch)(body)


_sc_agg_l1 = _make_sc_agg(with_deg=True)
_sc_agg_l2 = _make_sc_agg(with_deg=False)


ROW_BLK = 400
GRID = N_NODES // ROW_BLK


def _tc1_body(x_ref, sc_ref, deg_ref, ws_ref, wn_ref, b_ref, h_ref, inv_ref):
    agg = sc_ref[0] + sc_ref[1]
    deg = deg_ref[:, 0:1] + deg_ref[:, 1:2]
    inv = 1.0 / jnp.maximum(deg, 1.0)
    mean = agg * inv
    h = (jnp.dot(x_ref[...], ws_ref[...], preferred_element_type=jnp.float32)
         + jnp.dot(mean, wn_ref[...], preferred_element_type=jnp.float32)
         + b_ref[...])
    h_ref[...] = jax.nn.sigmoid(h)
    inv_ref[...] = jnp.broadcast_to(inv, (ROW_BLK, 8))


def _tc2_body(h1_ref, sc_ref, inv_ref, w2s_ref, w2n_ref, b2_ref,
              wm1_ref, bm1_ref, wm2_ref, bm2_ref, out_ref):
    agg = sc_ref[0] + sc_ref[1]
    inv = inv_ref[:, 0:1]
    h1 = h1_ref[...]
    h2 = jax.nn.sigmoid(
        jnp.dot(h1, w2s_ref[...], preferred_element_type=jnp.float32)
        + jnp.dot(agg * inv, w2n_ref[...], preferred_element_type=jnp.float32)
        + b2_ref[...])
    t = jnp.maximum(
        jnp.dot(h2, wm1_ref[...], preferred_element_type=jnp.float32)
        + bm1_ref[...], 0.0)
    out_ref[...] = (jnp.dot(t, wm2_ref[...], preferred_element_type=jnp.float32)
                    + bm2_ref[...])


def _full(shape):
    return pl.BlockSpec(shape, lambda i: tuple(0 for _ in shape))


def kernel(features, edge_index, W1_self, W1_neigh, b1, W2_self, W2_neigh, b2,
           Wm1, bm1, Wm2, bm2):
    src = jnp.asarray(edge_index[0], jnp.int32)
    dst = jnp.asarray(edge_index[1], jnp.int32)
    pad = EPAD - N_EDGES
    pad_i = jnp.arange(pad, dtype=jnp.int32)
    src_r = jnp.concatenate([src, pad_i % N_NODES]).reshape(IDX_ROWS, 128)
    dst_r = jnp.concatenate([dst, N_NODES + pad_i % (NPAD - N_NODES)]).reshape(
        IDX_ROWS, 128)

    z2 = jnp.zeros((NPAD, D_FEAT), jnp.float32)
    zd = jnp.zeros((NPAD,), jnp.float32)

    sc1, deg2 = _sc_agg_l1(features, src_r, dst_r, z2, zd)
    deg_pair = deg2[:, :N_NODES].T

    h1, inv8 = pl.pallas_call(
        _tc1_body,
        grid=(GRID,),
        in_specs=[
            pl.BlockSpec((ROW_BLK, D_FEAT), lambda i: (i, 0)),
            pl.BlockSpec((NC, ROW_BLK, D_FEAT), lambda i: (0, i, 0)),
            pl.BlockSpec((ROW_BLK, NC), lambda i: (i, 0)),
            _full((D_FEAT, D_FEAT)),
            _full((D_FEAT, D_FEAT)),
            _full((1, D_FEAT)),
        ],
        out_specs=[
            pl.BlockSpec((ROW_BLK, D_FEAT), lambda i: (i, 0)),
            pl.BlockSpec((ROW_BLK, 8), lambda i: (i, 0)),
        ],
        out_shape=[
            jax.ShapeDtypeStruct((N_NODES, D_FEAT), jnp.float32),
            jax.ShapeDtypeStruct((N_NODES, 8), jnp.float32),
        ],
    )(features, sc1[:, :N_NODES], deg_pair, W1_self, W1_neigh,
      b1.reshape(1, D_FEAT))

    sc2, = _sc_agg_l2(h1, src_r, dst_r, z2)

    out = pl.pallas_call(
        _tc2_body,
        grid=(GRID,),
        in_specs=[
            pl.BlockSpec((ROW_BLK, D_FEAT), lambda i: (i, 0)),
            pl.BlockSpec((NC, ROW_BLK, D_FEAT), lambda i: (0, i, 0)),
            pl.BlockSpec((ROW_BLK, 8), lambda i: (i, 0)),
            _full((D_FEAT, D_FEAT)),
            _full((D_FEAT, D_FEAT)),
            _full((1, D_FEAT)),
            _full((D_FEAT, D_FEAT)),
            _full((1, D_FEAT)),
            _full((D_FEAT, 64)),
            _full((1, 64)),
        ],
        out_specs=pl.BlockSpec((ROW_BLK, 64), lambda i: (i, 0)),
        out_shape=jax.ShapeDtypeStruct((N_NODES, 64), jnp.float32),
    )(h1, sc2[:, :N_NODES], inv8, W2_self, W2_neigh, b2.reshape(1, D_FEAT),
      Wm1, bm1.reshape(1, D_FEAT), Wm2, bm2.reshape(1, 64))

    return out


# speedup vs baseline: 10.4709x; 1.0635x over previous
import functools

import jax
import jax.numpy as jnp
from jax import lax
from jax.experimental import pallas as pl
from jax.experimental.pallas import tpu as pltpu
from jax.experimental.pallas import tpu_sc as plsc

N_NODES = 10000
N_EDGES = 320000
D_FEAT = 128

NC = 2
NS = 16
NW = NC * NS

NPAD = 10240
EPAD = 327680
IDX_ROWS = EPAD // 128
ROWS_PER_W = IDX_ROWS // NW
KI = 2
IB = 40
ROWS_PER_SUB = NPAD // NS


def _make_sc_agg(with_deg):
    mesh = plsc.VectorSubcoreMesh(core_axis_name="c", subcore_axis_name="s")

    out_type = [jax.ShapeDtypeStruct((NC, NPAD, D_FEAT), jnp.float32)]
    scratch = [
        pltpu.VMEM((IB, 128), jnp.int32),
        pltpu.VMEM((IB, 128), jnp.int32),
        pltpu.VMEM((KI, 128, D_FEAT), jnp.float32),
        pltpu.VMEM_SHARED((NPAD, D_FEAT), jnp.float32),
        pltpu.SemaphoreType.DMA,
        pltpu.SemaphoreType.DMA,
        pltpu.SemaphoreType.DMA,
        pltpu.SemaphoreType.DMA,
    ]
    if with_deg:
        out_type.append(jax.ShapeDtypeStruct((NC, NPAD), jnp.float32))
        scratch += [
            pltpu.VMEM((128,), jnp.float32),
            pltpu.VMEM_SHARED((NPAD,), jnp.float32),
            pltpu.SemaphoreType.DMA,
        ]

    NP = IB // 2

    def body(table, src_r, dst_r, zeros, *rest):
        if with_deg:
            (zerosd, out, out_deg, sidx, didx, rows, acc, gs0, gs1, ss0, ss1,
             ones, acc_deg, dsem) = rest
        else:
            out, sidx, didx, rows, acc, gs0, gs1, ss0, ss1 = rest

        c = lax.axis_index("c")
        s = lax.axis_index("s")
        wid = c * NS + s

        gsem = (gs0, gs1)
        ssem = (ss0, ss1)

        zsl = pl.ds(s * ROWS_PER_SUB, ROWS_PER_SUB)
        pltpu.sync_copy(zeros.at[zsl], acc.at[zsl])
        if with_deg:
            pltpu.sync_copy(zerosd.at[zsl], acc_deg.at[zsl])
            for i in range(8):
                ones[pl.ds(16 * i, 16)] = jnp.ones((16,), jnp.float32)
        plsc.subcore_barrier()

        base = wid * ROWS_PER_W

        def gather(i, b):
            return pltpu.async_copy(table.at[sidx.at[i]], rows.at[b], gsem[b])

        def scatter(i, b):
            pltpu.async_copy(rows.at[b], acc.at[didx.at[i]], ssem[b], add=True)
            if with_deg:
                pltpu.async_copy(ones, acc_deg.at[didx.at[i]], dsem, add=True)

        def wait_gather(i, b):
            pltpu.make_async_copy(table.at[sidx.at[i]], rows.at[b],
                                  gsem[b]).wait()

        def wait_scatter(i, b):
            pltpu.make_async_copy(rows.at[b], acc.at[didx.at[i]],
                                  ssem[b]).wait()

        def outer(ob, carry):
            r0 = base + ob * IB
            pltpu.sync_copy(src_r.at[pl.ds(r0, IB)], sidx)
            pltpu.sync_copy(dst_r.at[pl.ds(r0, IB)], didx)

            gather(0, 0)
            gather(1, 1)

            def pair(gp, c2):
                i0 = gp * 2
                wait_gather(i0, 0)
                scatter(i0, 0)
                wait_gather(i0 + 1, 1)
                scatter(i0 + 1, 1)
                wait_scatter(i0, 0)
                gather(i0 + 2, 0)
                wait_scatter(i0 + 1, 1)
                gather(i0 + 3, 1)
                return c2

            lax.fori_loop(0, NP - 1, pair, carry)

            i0 = (NP - 1) * 2
            wait_gather(i0, 0)
            scatter(i0, 0)
            wait_gather(i0 + 1, 1)
            scatter(i0 + 1, 1)
            wait_scatter(i0, 0)
            wait_scatter(i0 + 1, 1)
            if with_deg:
                def drain(i, c3):
                    pltpu.make_async_copy(ones, acc_deg.at[didx.at[0]],
                                          dsem).wait()
                    return c3
                lax.fori_loop(0, IB, drain, carry)
            return carry

        lax.fori_loop(0, ROWS_PER_W // IB, outer, 0)

        plsc.subcore_barrier()
        pltpu.sync_copy(acc.at[zsl], out.at[c, zsl])
        if with_deg:
            pltpu.sync_copy(acc_deg.at[zsl], out_deg.at[c, zsl])

    return functools.partial(
        pl.kernel, mesh=mesh, out_type=out_type, scratch_types=scratch)(body)


_sc_agg_l1 = _make_sc_agg(with_deg=True)
_sc_agg_l2 = _make_sc_agg(with_deg=False)


ROW_BLK = 2000
GRID = N_NODES // ROW_BLK


def _tc1_body(x_ref, sc_ref, deg_ref, ws_ref, wn_ref, b_ref, h_ref, inv_ref):
    agg = sc_ref[0] + sc_ref[1]
    deg = deg_ref[:, 0:1] + deg_ref[:, 1:2]
    inv = 1.0 / jnp.maximum(deg, 1.0)
    mean = agg * inv
    h = (jnp.dot(x_ref[...], ws_ref[...], preferred_element_type=jnp.float32)
         + jnp.dot(mean, wn_ref[...], preferred_element_type=jnp.float32)
         + b_ref[...])
    h_ref[...] = jax.nn.sigmoid(h)
    inv_ref[...] = jnp.broadcast_to(inv, (ROW_BLK, 8))


def _tc2_body(h1_ref, sc_ref, inv_ref, w2s_ref, w2n_ref, b2_ref,
              wm1_ref, bm1_ref, wm2_ref, bm2_ref, out_ref):
    agg = sc_ref[0] + sc_ref[1]
    inv = inv_ref[:, 0:1]
    h1 = h1_ref[...]
    h2 = jax.nn.sigmoid(
        jnp.dot(h1, w2s_ref[...], preferred_element_type=jnp.float32)
        + jnp.dot(agg * inv, w2n_ref[...], preferred_element_type=jnp.float32)
        + b2_ref[...])
    t = jnp.maximum(
        jnp.dot(h2, wm1_ref[...], preferred_element_type=jnp.float32)
        + bm1_ref[...], 0.0)
    out_ref[...] = (jnp.dot(t, wm2_ref[...], preferred_element_type=jnp.float32)
                    + bm2_ref[...])


def _full(shape):
    return pl.BlockSpec(shape, lambda i: tuple(0 for _ in shape))


def kernel(features, edge_index, W1_self, W1_neigh, b1, W2_self, W2_neigh, b2,
           Wm1, bm1, Wm2, bm2):
    src = jnp.asarray(edge_index[0], jnp.int32)
    dst = jnp.asarray(edge_index[1], jnp.int32)
    pad = EPAD - N_EDGES
    pad_i = jnp.arange(pad, dtype=jnp.int32)
    src_r = jnp.concatenate([src, pad_i % N_NODES]).reshape(IDX_ROWS, 128)
    dst_r = jnp.concatenate([dst, N_NODES + pad_i % (NPAD - N_NODES)]).reshape(
        IDX_ROWS, 128)

    z2 = jnp.zeros((NPAD, D_FEAT), jnp.float32)
    zd = jnp.zeros((NPAD,), jnp.float32)

    sc1, deg2 = _sc_agg_l1(features, src_r, dst_r, z2, zd)
    deg_pair = deg2[:, :N_NODES].T

    h1, inv8 = pl.pallas_call(
        _tc1_body,
        grid=(GRID,),
        in_specs=[
            pl.BlockSpec((ROW_BLK, D_FEAT), lambda i: (i, 0)),
            pl.BlockSpec((NC, ROW_BLK, D_FEAT), lambda i: (0, i, 0)),
            pl.BlockSpec((ROW_BLK, NC), lambda i: (i, 0)),
            _full((D_FEAT, D_FEAT)),
            _full((D_FEAT, D_FEAT)),
            _full((1, D_FEAT)),
        ],
        out_specs=[
            pl.BlockSpec((ROW_BLK, D_FEAT), lambda i: (i, 0)),
            pl.BlockSpec((ROW_BLK, 8), lambda i: (i, 0)),
        ],
        out_shape=[
            jax.ShapeDtypeStruct((N_NODES, D_FEAT), jnp.float32),
            jax.ShapeDtypeStruct((N_NODES, 8), jnp.float32),
        ],
    )(features, sc1[:, :N_NODES], deg_pair, W1_self, W1_neigh,
      b1.reshape(1, D_FEAT))

    sc2, = _sc_agg_l2(h1, src_r, dst_r, z2)

    out = pl.pallas_call(
        _tc2_body,
        grid=(GRID,),
        in_specs=[
            pl.BlockSpec((ROW_BLK, D_FEAT), lambda i: (i, 0)),
            pl.BlockSpec((NC, ROW_BLK, D_FEAT), lambda i: (0, i, 0)),
            pl.BlockSpec((ROW_BLK, 8), lambda i: (i, 0)),
            _full((D_FEAT, D_FEAT)),
            _full((D_FEAT, D_FEAT)),
            _full((1, D_FEAT)),
            _full((D_FEAT, D_FEAT)),
            _full((1, D_FEAT)),
            _full((D_FEAT, 64)),
            _full((1, 64)),
        ],
        out_specs=pl.BlockSpec((ROW_BLK, 64), lambda i: (i, 0)),
        out_shape=jax.ShapeDtypeStruct((N_NODES, 64), jnp.float32),
    )(h1, sc2[:, :N_NODES], inv8, W2_self, W2_neigh, b2.reshape(1, D_FEAT),
      Wm1, bm1.reshape(1, D_FEAT), Wm2, bm2.reshape(1, 64))

    return out


# speedup vs baseline: 10.9179x; 1.0427x over previous
import functools

import jax
import jax.numpy as jnp
from jax import lax
from jax.experimental import pallas as pl
from jax.experimental.pallas import tpu as pltpu
from jax.experimental.pallas import tpu_sc as plsc

N_NODES = 10000
N_EDGES = 320000
D_FEAT = 128

NC = 2
NS = 16
NW = NC * NS

NPAD = 10240
EPAD = 327680
IDX_ROWS = EPAD // 128
ROWS_PER_W = IDX_ROWS // NW
KI = 2
IB = 40
ROWS_PER_SUB = NPAD // NS


def _make_sc_agg(with_deg):
    mesh = plsc.VectorSubcoreMesh(core_axis_name="c", subcore_axis_name="s")

    out_type = [jax.ShapeDtypeStruct((NC, NPAD, D_FEAT), jnp.float32)]
    scratch = [
        pltpu.VMEM((IB, 128), jnp.int32),
        pltpu.VMEM((IB, 128), jnp.int32),
        pltpu.VMEM((KI, 128, D_FEAT), jnp.float32),
        pltpu.VMEM_SHARED((NPAD, D_FEAT), jnp.float32),
        pltpu.SemaphoreType.DMA,
        pltpu.SemaphoreType.DMA,
        pltpu.SemaphoreType.DMA,
        pltpu.SemaphoreType.DMA,
    ]
    if with_deg:
        out_type.append(jax.ShapeDtypeStruct((NC, NPAD), jnp.float32))
        scratch += [
            pltpu.VMEM((128,), jnp.float32),
            pltpu.VMEM_SHARED((NPAD,), jnp.float32),
            pltpu.SemaphoreType.DMA,
        ]

    NP = IB // 2

    def body(table, src_r, dst_r, zeros, *rest):
        if with_deg:
            (zerosd, out, out_deg, sidx, didx, rows, acc, gs0, gs1, ss0, ss1,
             ones, acc_deg, dsem) = rest
        else:
            out, sidx, didx, rows, acc, gs0, gs1, ss0, ss1 = rest

        c = lax.axis_index("c")
        s = lax.axis_index("s")
        wid = c * NS + s

        gsem = (gs0, gs1)
        ssem = (ss0, ss1)

        zsl = pl.ds(s * ROWS_PER_SUB, ROWS_PER_SUB)
        pltpu.sync_copy(zeros.at[zsl], acc.at[zsl])
        if with_deg:
            pltpu.sync_copy(zerosd.at[zsl], acc_deg.at[zsl])
            for i in range(8):
                ones[pl.ds(16 * i, 16)] = jnp.ones((16,), jnp.float32)
        plsc.subcore_barrier()

        base = wid * ROWS_PER_W

        def gather(i, b):
            return pltpu.async_copy(table.at[sidx.at[i]], rows.at[b], gsem[b])

        def scatter(i, b):
            pltpu.async_copy(rows.at[b], acc.at[didx.at[i]], ssem[b], add=True)
            if with_deg:
                pltpu.async_copy(ones, acc_deg.at[didx.at[i]], dsem, add=True)

        def wait_gather(i, b):
            pltpu.make_async_copy(table.at[sidx.at[i]], rows.at[b],
                                  gsem[b]).wait()

        def wait_scatter(i, b):
            pltpu.make_async_copy(rows.at[b], acc.at[didx.at[i]],
                                  ssem[b]).wait()

        def outer(ob, carry):
            r0 = base + ob * IB
            pltpu.sync_copy(src_r.at[pl.ds(r0, IB)], sidx)
            pltpu.sync_copy(dst_r.at[pl.ds(r0, IB)], didx)

            gather(0, 0)
            gather(1, 1)

            def pair(gp, c2):
                i0 = gp * 2
                wait_gather(i0, 0)
                scatter(i0, 0)
                wait_gather(i0 + 1, 1)
                scatter(i0 + 1, 1)
                wait_scatter(i0, 0)
                gather(i0 + 2, 0)
                wait_scatter(i0 + 1, 1)
                gather(i0 + 3, 1)
                return c2

            lax.fori_loop(0, NP - 1, pair, carry)

            i0 = (NP - 1) * 2
            wait_gather(i0, 0)
            scatter(i0, 0)
            wait_gather(i0 + 1, 1)
            scatter(i0 + 1, 1)
            wait_scatter(i0, 0)
            wait_scatter(i0 + 1, 1)
            if with_deg:
                def drain(i, c3):
                    pltpu.make_async_copy(ones, acc_deg.at[didx.at[0]],
                                          dsem).wait()
                    return c3
                lax.fori_loop(0, IB, drain, carry)
            return carry

        lax.fori_loop(0, ROWS_PER_W // IB, outer, 0)

        plsc.subcore_barrier()
        pltpu.sync_copy(acc.at[zsl], out.at[c, zsl])
        if with_deg:
            pltpu.sync_copy(acc_deg.at[zsl], out_deg.at[c, zsl])

    return functools.partial(
        pl.kernel, mesh=mesh, out_type=out_type, scratch_types=scratch)(body)


_sc_agg_l1 = _make_sc_agg(with_deg=True)
_sc_agg_l2 = _make_sc_agg(with_deg=False)


ROW_BLK = 2048
GRID = NPAD // ROW_BLK


def _tc0_body(deg_ref, inv_ref):
    d = deg_ref[0:1, :] + deg_ref[1:2, :]
    inv_ref[...] = 1.0 / jnp.maximum(d, 1.0)


def _tc1_body(x_ref, sc_ref, inv_ref, ws_ref, wn_ref, b_ref, h_ref):
    agg = sc_ref[0] + sc_ref[1]
    mean = agg * inv_ref[...]
    h = (jnp.dot(x_ref[...], ws_ref[...], preferred_element_type=jnp.float32)
         + jnp.dot(mean, wn_ref[...], preferred_element_type=jnp.float32)
         + b_ref[...])
    h_ref[...] = jax.nn.sigmoid(h)


def _tc2_body(h1_ref, sc_ref, inv_ref, w2s_ref, w2n_ref, b2_ref,
              wm1_ref, bm1_ref, wm2_ref, bm2_ref, out_ref):
    agg = sc_ref[0] + sc_ref[1]
    h1 = h1_ref[...]
    h2 = jax.nn.sigmoid(
        jnp.dot(h1, w2s_ref[...], preferred_element_type=jnp.float32)
        + jnp.dot(agg * inv_ref[...], w2n_ref[...],
                  preferred_element_type=jnp.float32)
        + b2_ref[...])
    t = jnp.maximum(
        jnp.dot(h2, wm1_ref[...], preferred_element_type=jnp.float32)
        + bm1_ref[...], 0.0)
    out_ref[...] = (jnp.dot(t, wm2_ref[...], preferred_element_type=jnp.float32)
                    + bm2_ref[...])


def _full(shape):
    return pl.BlockSpec(shape, lambda i: tuple(0 for _ in shape))


def kernel(features, edge_index, W1_self, W1_neigh, b1, W2_self, W2_neigh, b2,
           Wm1, bm1, Wm2, bm2):
    src = jnp.asarray(edge_index[0], jnp.int32)
    dst = jnp.asarray(edge_index[1], jnp.int32)
    pad = EPAD - N_EDGES
    pad_i = jnp.arange(pad, dtype=jnp.int32)
    src_r = jnp.concatenate([src, pad_i % N_NODES]).reshape(IDX_ROWS, 128)
    dst_r = jnp.concatenate([dst, N_NODES + pad_i % (NPAD - N_NODES)]).reshape(
        IDX_ROWS, 128)

    z2 = jnp.zeros((NPAD, D_FEAT), jnp.float32)
    zd = jnp.zeros((NPAD,), jnp.float32)

    sc1, deg2 = _sc_agg_l1(features, src_r, dst_r, z2, zd)

    inv_row = pl.pallas_call(
        _tc0_body,
        grid=(1,),
        in_specs=[_full((NC, NPAD))],
        out_specs=_full((1, NPAD)),
        out_shape=jax.ShapeDtypeStruct((1, NPAD), jnp.float32),
    )(deg2)
    inv_col = inv_row.reshape(NPAD, 1)

    h1 = pl.pallas_call(
        _tc1_body,
        grid=(GRID,),
        in_specs=[
            pl.BlockSpec((ROW_BLK, D_FEAT), lambda i: (i, 0)),
            pl.BlockSpec((NC, ROW_BLK, D_FEAT), lambda i: (0, i, 0)),
            pl.BlockSpec((ROW_BLK, 1), lambda i: (i, 0)),
            _full((D_FEAT, D_FEAT)),
            _full((D_FEAT, D_FEAT)),
            _full((1, D_FEAT)),
        ],
        out_specs=pl.BlockSpec((ROW_BLK, D_FEAT), lambda i: (i, 0)),
        out_shape=jax.ShapeDtypeStruct((N_NODES, D_FEAT), jnp.float32),
    )(features, sc1, inv_col, W1_self, W1_neigh, b1.reshape(1, D_FEAT))

    sc2, = _sc_agg_l2(h1, src_r, dst_r, z2)

    out = pl.pallas_call(
        _tc2_body,
        grid=(GRID,),
        in_specs=[
            pl.BlockSpec((ROW_BLK, D_FEAT), lambda i: (i, 0)),
            pl.BlockSpec((NC, ROW_BLK, D_FEAT), lambda i: (0, i, 0)),
            pl.BlockSpec((ROW_BLK, 1), lambda i: (i, 0)),
            _full((D_FEAT, D_FEAT)),
            _full((D_FEAT, D_FEAT)),
            _full((1, D_FEAT)),
            _full((D_FEAT, D_FEAT)),
            _full((1, D_FEAT)),
            _full((D_FEAT, 64)),
            _full((1, 64)),
        ],
        out_specs=pl.BlockSpec((ROW_BLK, 64), lambda i: (i, 0)),
        out_shape=jax.ShapeDtypeStruct((N_NODES, 64), jnp.float32),
    )(h1, sc2, inv_col, W2_self, W2_neigh, b2.reshape(1, D_FEAT),
      Wm1, bm1.reshape(1, D_FEAT), Wm2, bm2.reshape(1, 64))

    return out


# speedup vs baseline: 11.9667x; 1.0961x over previous
import functools

import jax
import jax.numpy as jnp
from jax import lax
from jax.experimental import pallas as pl
from jax.experimental.pallas import tpu as pltpu
from jax.experimental.pallas import tpu_sc as plsc

N_NODES = 10000
N_EDGES = 320000
D_FEAT = 128

NC = 2
NS = 16
NW = NC * NS

NPAD = 10240
EPAD = 327680
CHUNK = 64
IDX_ROWS = EPAD // CHUNK
ROWS_PER_W = IDX_ROWS // NW
NB = 4
IB = 40
ROWS_PER_SUB = NPAD // NS


def _make_sc_agg(with_deg):
    mesh = plsc.VectorSubcoreMesh(core_axis_name="c", subcore_axis_name="s")

    out_type = [jax.ShapeDtypeStruct((NC, NPAD, D_FEAT), jnp.float32)]
    scratch = [
        pltpu.VMEM((IB, CHUNK), jnp.int32),
        pltpu.VMEM((IB, CHUNK), jnp.int32),
        pltpu.VMEM((NB, CHUNK, D_FEAT), jnp.float32),
        pltpu.VMEM_SHARED((NPAD, D_FEAT), jnp.float32),
    ] + [pltpu.SemaphoreType.DMA] * (2 * NB)
    if with_deg:
        out_type.append(jax.ShapeDtypeStruct((NC, NPAD), jnp.float32))
        scratch += [
            pltpu.VMEM((CHUNK,), jnp.float32),
            pltpu.VMEM_SHARED((NPAD,), jnp.float32),
            pltpu.SemaphoreType.DMA,
        ]

    NP = IB // NB

    def body(table, src_r, dst_r, zeros, *rest):
        if with_deg:
            (zerosd, out, out_deg, sidx, didx, rows, acc, *sems) = rest
            gsem, ssem = sems[:NB], sems[NB:2 * NB]
            ones, acc_deg, dsem = sems[2 * NB:]
        else:
            out, sidx, didx, rows, acc, *sems = rest
            gsem, ssem = sems[:NB], sems[NB:2 * NB]

        c = lax.axis_index("c")
        s = lax.axis_index("s")
        wid = c * NS + s

        zsl = pl.ds(s * ROWS_PER_SUB, ROWS_PER_SUB)
        pltpu.sync_copy(zeros.at[zsl], acc.at[zsl])
        if with_deg:
            pltpu.sync_copy(zerosd.at[zsl], acc_deg.at[zsl])
            for i in range(CHUNK // 16):
                ones[pl.ds(16 * i, 16)] = jnp.ones((16,), jnp.float32)
        plsc.subcore_barrier()

        base = wid * ROWS_PER_W

        def gather(i, b):
            pltpu.async_copy(table.at[sidx.at[i]], rows.at[b], gsem[b])

        def scatter(i, b):
            pltpu.async_copy(rows.at[b], acc.at[didx.at[i]], ssem[b], add=True)
            if with_deg:
                pltpu.async_copy(ones, acc_deg.at[didx.at[i]], dsem, add=True)

        def wait_gather(i, b):
            pltpu.make_async_copy(table.at[sidx.at[i]], rows.at[b],
                                  gsem[b]).wait()

        def wait_scatter(i, b):
            pltpu.make_async_copy(rows.at[b], acc.at[didx.at[i]],
                                  ssem[b]).wait()

        def outer(ob, carry):
            r0 = base + ob * IB
            pltpu.sync_copy(src_r.at[pl.ds(r0, IB)], sidx)
            pltpu.sync_copy(dst_r.at[pl.ds(r0, IB)], didx)

            for b in range(NB):
                gather(b, b)

            def quad(gp, c2):
                i0 = gp * NB
                for b in range(NB):
                    wait_gather(i0 + b, b)
                    scatter(i0 + b, b)
                for b in range(NB):
                    wait_scatter(i0 + b, b)
                    gather(i0 + NB + b, b)
                return c2

            lax.fori_loop(0, NP - 1, quad, carry)

            i0 = (NP - 1) * NB
            for b in range(NB):
                wait_gather(i0 + b, b)
                scatter(i0 + b, b)
            for b in range(NB):
                wait_scatter(i0 + b, b)
            if with_deg:
                def drain(i, c3):
                    pltpu.make_async_copy(ones, acc_deg.at[didx.at[0]],
                                          dsem).wait()
                    return c3
                lax.fori_loop(0, IB, drain, carry)
            return carry

        lax.fori_loop(0, ROWS_PER_W // IB, outer, 0)

        plsc.subcore_barrier()
        pltpu.sync_copy(acc.at[zsl], out.at[c, zsl])
        if with_deg:
            pltpu.sync_copy(acc_deg.at[zsl], out_deg.at[c, zsl])

    return functools.partial(
        pl.kernel, mesh=mesh, out_type=out_type, scratch_types=scratch)(body)


_sc_agg_l1 = _make_sc_agg(with_deg=True)
_sc_agg_l2 = _make_sc_agg(with_deg=False)


ROW_BLK = 2048
GRID = NPAD // ROW_BLK


def _tc0_body(deg_ref, inv_ref):
    d = deg_ref[0:1, :] + deg_ref[1:2, :]
    inv_ref[...] = 1.0 / jnp.maximum(d, 1.0)


def _tc1_body(x_ref, sc_ref, inv_ref, ws_ref, wn_ref, b_ref, h_ref):
    agg = sc_ref[0] + sc_ref[1]
    mean = agg * inv_ref[...]
    h = (jnp.dot(x_ref[...], ws_ref[...], preferred_element_type=jnp.float32)
         + jnp.dot(mean, wn_ref[...], preferred_element_type=jnp.float32)
         + b_ref[...])
    h_ref[...] = jax.nn.sigmoid(h)


def _tc2_body(h1_ref, sc_ref, inv_ref, w2s_ref, w2n_ref, b2_ref,
              wm1_ref, bm1_ref, wm2_ref, bm2_ref, out_ref):
    agg = sc_ref[0] + sc_ref[1]
    h1 = h1_ref[...]
    h2 = jax.nn.sigmoid(
        jnp.dot(h1, w2s_ref[...], preferred_element_type=jnp.float32)
        + jnp.dot(agg * inv_ref[...], w2n_ref[...],
                  preferred_element_type=jnp.float32)
        + b2_ref[...])
    t = jnp.maximum(
        jnp.dot(h2, wm1_ref[...], preferred_element_type=jnp.float32)
        + bm1_ref[...], 0.0)
    out_ref[...] = (jnp.dot(t, wm2_ref[...], preferred_element_type=jnp.float32)
                    + bm2_ref[...])


def _full(shape):
    return pl.BlockSpec(shape, lambda i: tuple(0 for _ in shape))


def kernel(features, edge_index, W1_self, W1_neigh, b1, W2_self, W2_neigh, b2,
           Wm1, bm1, Wm2, bm2):
    src = jnp.asarray(edge_index[0], jnp.int32)
    dst = jnp.asarray(edge_index[1], jnp.int32)
    pad = EPAD - N_EDGES
    pad_i = jnp.arange(pad, dtype=jnp.int32)
    src_r = jnp.concatenate([src, pad_i % N_NODES]).reshape(IDX_ROWS, CHUNK)
    dst_r = jnp.concatenate([dst, N_NODES + pad_i % (NPAD - N_NODES)]).reshape(
        IDX_ROWS, CHUNK)

    z2 = jnp.zeros((NPAD, D_FEAT), jnp.float32)
    zd = jnp.zeros((NPAD,), jnp.float32)

    sc1, deg2 = _sc_agg_l1(features, src_r, dst_r, z2, zd)

    inv_row = pl.pallas_call(
        _tc0_body,
        grid=(1,),
        in_specs=[_full((NC, NPAD))],
        out_specs=_full((1, NPAD)),
        out_shape=jax.ShapeDtypeStruct((1, NPAD), jnp.float32),
    )(deg2)
    inv_col = inv_row.reshape(NPAD, 1)

    h1 = pl.pallas_call(
        _tc1_body,
        grid=(GRID,),
        in_specs=[
            pl.BlockSpec((ROW_BLK, D_FEAT), lambda i: (i, 0)),
            pl.BlockSpec((NC, ROW_BLK, D_FEAT), lambda i: (0, i, 0)),
            pl.BlockSpec((ROW_BLK, 1), lambda i: (i, 0)),
            _full((D_FEAT, D_FEAT)),
            _full((D_FEAT, D_FEAT)),
            _full((1, D_FEAT)),
        ],
        out_specs=pl.BlockSpec((ROW_BLK, D_FEAT), lambda i: (i, 0)),
        out_shape=jax.ShapeDtypeStruct((N_NODES, D_FEAT), jnp.float32),
    )(features, sc1, inv_col, W1_self, W1_neigh, b1.reshape(1, D_FEAT))

    sc2, = _sc_agg_l2(h1, src_r, dst_r, z2)

    out = pl.pallas_call(
        _tc2_body,
        grid=(GRID,),
        in_specs=[
            pl.BlockSpec((ROW_BLK, D_FEAT), lambda i: (i, 0)),
            pl.BlockSpec((NC, ROW_BLK, D_FEAT), lambda i: (0, i, 0)),
            pl.BlockSpec((ROW_BLK, 1), lambda i: (i, 0)),
            _full((D_FEAT, D_FEAT)),
            _full((D_FEAT, D_FEAT)),
            _full((1, D_FEAT)),
            _full((D_FEAT, D_FEAT)),
            _full((1, D_FEAT)),
            _full((D_FEAT, 64)),
            _full((1, 64)),
        ],
        out_specs=pl.BlockSpec((ROW_BLK, 64), lambda i: (i, 0)),
        out_shape=jax.ShapeDtypeStruct((N_NODES, 64), jnp.float32),
    )(h1, sc2, inv_col, W2_self, W2_neigh, b2.reshape(1, D_FEAT),
      Wm1, bm1.reshape(1, D_FEAT), Wm2, bm2.reshape(1, 64))

    return out
